# Initial kernel scaffold; baseline (speedup 1.0000x reference)
import jax
import jax.numpy as jnp
from jax.experimental import pallas as pl


def kernel(y_pred, y_true):
    raise NotImplementedError("write your pallas kernel here")


# speedup vs baseline: 1.0491x; 1.0491x over previous
import functools

import jax
import jax.numpy as jnp
from jax.experimental import pallas as pl
from jax.experimental.pallas import tpu as pltpu

TEMPERATURE = 10.0
TOP_FRACTION = 0.1
LAMBDA_TOPK = 50
IC_W = 0.5

_TILE = 256


def _rank_kernel(yp_ref, yt_ref, ypt_ref, ytt_ref, rt_ref, rp_ref):
    i = pl.program_id(0)
    n = yp_ref.shape[1]
    tile = ypt_ref.shape[0]
    col_idx = jax.lax.broadcasted_iota(jnp.int32, (1, n), 1)
    row_idx = i * tile + jax.lax.broadcasted_iota(jnp.int32, (tile, 1), 0)
    earlier = col_idx < row_idx

    yp_c = yp_ref[...]
    yt_c = yt_ref[...]
    yp_r = ypt_ref[...]
    yt_r = ytt_ref[...]

    cmp_t = (yt_c > yt_r) | ((yt_c == yt_r) & earlier)
    rt_ref[...] = jnp.sum(cmp_t.astype(jnp.int32), axis=1, keepdims=True)
    cmp_p = (yp_c > yp_r) | ((yp_c == yp_r) & earlier)
    rp_ref[...] = jnp.sum(cmp_p.astype(jnp.int32), axis=1, keepdims=True)


def _softplus(x):
    return jnp.maximum(x, 0.0) + jnp.log1p(jnp.exp(-jnp.abs(x)))


def _loss_kernel(k_pair, k_lam, nt,
                 yp_ref, yt_ref, ypt_ref, ytt_ref,
                 rtr_ref, rpr_ref, rtc_ref, rpc_ref,
                 out_ref, acc_ref):
    i = pl.program_id(0)
    n = yp_ref.shape[1]

    @pl.when(i == 0)
    def _init():
        acc_ref[0] = 0.0
        acc_ref[1] = 0.0
        acc_ref[2] = 0.0
        acc_ref[3] = 0.0

    yt_c = yt_ref[...]
    yp_c = yp_ref[...]
    mn = jnp.min(yt_c)
    mx = jnp.max(yt_c)
    denom = (mx - mn) + 1e-8
    g_c = (yt_c - mn) / denom
    d_c = 1.0 / jnp.log2(rpr_ref[...].astype(jnp.float32) + 2.0)

    yp_r = ypt_ref[...]
    yt_r = ytt_ref[...]
    g_r = (yt_r - mn) / denom
    d_r = 1.0 / jnp.log2(rpc_ref[...].astype(jnp.float32) + 2.0)
    rt_r = rtc_ref[...]
    top_pair = rt_r < k_pair
    top_lam = rt_r < k_lam

    x = yp_c - yp_r
    sp = _softplus(x)
    td = yt_r - yt_c
    pos = td > 0.0
    m_pair = pos & top_pair
    m_lam = pos & top_lam

    pair_num = jnp.sum(jnp.where(m_pair, sp * td, 0.0))
    pair_cnt = jnp.sum(m_pair.astype(jnp.float32))
    delta = jnp.abs((g_r - g_c) * (d_r - d_c))
    lam_num = jnp.sum(jnp.where(m_lam, sp * delta, 0.0))
    lam_cnt = jnp.sum(m_lam.astype(jnp.float32))

    acc_ref[0] += pair_num
    acc_ref[1] += pair_cnt
    acc_ref[2] += lam_num
    acc_ref[3] += lam_cnt

    @pl.when(i == nt - 1)
    def _final():
        ideal_dcg = jnp.sum(
            g_c / jnp.log2(rtr_ref[...].astype(jnp.float32) + 2.0)) + 1e-8

        a = yt_c * TEMPERATURE
        e = jnp.exp(a - jnp.max(a))
        p_true = e / jnp.sum(e)
        bm = jnp.max(yp_c)
        logsm = (yp_c - bm) - jnp.log(jnp.sum(jnp.exp(yp_c - bm)))
        listnet = -jnp.sum(p_true * logsm)

        xc = yp_c - jnp.mean(yp_c)
        yc = yt_c - jnp.mean(yt_c)
        xs = jnp.sqrt(jnp.mean(xc * xc) + 1e-12)
        ys = jnp.sqrt(jnp.mean(yc * yc) + 1e-12)
        corr = jnp.clip(jnp.mean(xc * yc) / (xs * ys + 1e-12), -1.0, 1.0)

        pair_loss = acc_ref[0] / (acc_ref[1] + 1e-8)
        cnt = acc_ref[3]
        lam_loss = jnp.where(cnt > 0.0,
                             (acc_ref[2] / ideal_dcg) / jnp.maximum(cnt, 1.0),
                             0.0)
        total = listnet + pair_loss + lam_loss + IC_W * (-corr)
        out_ref[...] = jnp.full((1, 1), total, dtype=jnp.float32)


def kernel(y_pred, y_true):
    n = y_pred.shape[1]
    tile = _TILE
    nt = n // tile
    k_pair = max(1, int(n * TOP_FRACTION))
    k_lam = min(LAMBDA_TOPK, n)

    yp = y_pred.reshape(1, n)
    yt = y_true.reshape(1, n)
    ypt = y_pred.reshape(n, 1)
    ytt = y_true.reshape(n, 1)

    full = pl.BlockSpec((1, n), lambda i: (0, 0))
    rows = pl.BlockSpec((tile, 1), lambda i: (i, 0))

    rt_col, rp_col = pl.pallas_call(
        _rank_kernel,
        grid=(nt,),
        in_specs=[full, full, rows, rows],
        out_specs=[rows, rows],
        out_shape=[jax.ShapeDtypeStruct((n, 1), jnp.int32),
                   jax.ShapeDtypeStruct((n, 1), jnp.int32)],
    )(yp, yt, ypt, ytt)

    rt_row = rt_col.reshape(1, n)
    rp_row = rp_col.reshape(1, n)

    out = pl.pallas_call(
        functools.partial(_loss_kernel, k_pair, k_lam, nt),
        grid=(nt,),
        in_specs=[full, full, rows, rows, full, full, rows, rows],
        out_specs=pl.BlockSpec((1, 1), lambda i: (0, 0)),
        out_shape=jax.ShapeDtypeStruct((1, 1), jnp.float32),
        scratch_shapes=[pltpu.SMEM((4,), jnp.float32)],
    )(yp, yt, ypt, ytt, rt_row, rp_row, rt_col, rp_col)

    return out[0, 0]


# speedup vs baseline: 2.4773x; 2.3613x over previous
import functools

import jax
import jax.numpy as jnp
from jax.experimental import pallas as pl
from jax.experimental.pallas import tpu as pltpu

TEMPERATURE = 10.0
TOP_FRACTION = 0.1
LAMBDA_TOPK = 50
IC_W = 0.5

_TILE_C = 512
_TILE_R = 256


def _softplus(x):
    return jnp.maximum(x, 0.0) + jnp.log1p(jnp.exp(-jnp.abs(x)))


def _fused_kernel(nt_rank, nt_loss, nc, k_pair, k_lam,
                  yp_ref, yt_ref, ypt_ref, ytt_ref, out_ref,
                  rtr_s, rpr_s, ypc_s, ytc_s, gr_s, dr_s, acc_ref):
    i = pl.program_id(0)
    n = yp_ref.shape[1]

    @pl.when(i < nt_rank)
    def _ranks():
        c0 = i * _TILE_C
        yp_cols = yp_ref[0:1, pl.ds(c0, _TILE_C)]
        yt_cols = yt_ref[0:1, pl.ds(c0, _TILE_C)]
        yp_rows = ypt_ref[...]
        yt_rows = ytt_ref[...]
        row_idx = jax.lax.broadcasted_iota(jnp.int32, (n, 1), 0)
        col_idx = c0 + jax.lax.broadcasted_iota(jnp.int32, (1, _TILE_C), 1)
        earlier = row_idx < col_idx
        cmp_t = (yt_rows > yt_cols) | ((yt_rows == yt_cols) & earlier)
        rtr_s[0:1, pl.ds(c0, _TILE_C)] = jnp.sum(
            cmp_t.astype(jnp.float32), axis=0, keepdims=True)
        cmp_p = (yp_rows > yp_cols) | ((yp_rows == yp_cols) & earlier)
        rpr_s[0:1, pl.ds(c0, _TILE_C)] = jnp.sum(
            cmp_p.astype(jnp.float32), axis=0, keepdims=True)

    @pl.when(i == nt_rank)
    def _compact():
        yt_c = yt_ref[...]
        yp_c = yp_ref[...]
        mn = jnp.min(yt_c)
        mx = jnp.max(yt_c)
        denom = (mx - mn) + 1e-8
        g_c = (yt_c - mn) / denom
        rtr = rtr_s[...]
        ideal = jnp.sum(g_c / jnp.log2(rtr + 2.0)) + 1e-8

        a = yt_c * TEMPERATURE
        e = jnp.exp(a - jnp.max(a))
        p_true = e / jnp.sum(e)
        bm = jnp.max(yp_c)
        logsm = (yp_c - bm) - jnp.log(jnp.sum(jnp.exp(yp_c - bm)))
        listnet = -jnp.sum(p_true * logsm)

        xc = yp_c - jnp.mean(yp_c)
        yc = yt_c - jnp.mean(yt_c)
        xs = jnp.sqrt(jnp.mean(xc * xc) + 1e-12)
        ys = jnp.sqrt(jnp.mean(yc * yc) + 1e-12)
        corr = jnp.clip(jnp.mean(xc * yc) / (xs * ys + 1e-12), -1.0, 1.0)

        r_iota = jax.lax.broadcasted_iota(
            jnp.int32, (nc, 1), 0).astype(jnp.float32)
        onehot = rtr == r_iota
        ypc = jnp.sum(jnp.where(onehot, yp_c, 0.0), axis=1, keepdims=True)
        ytc = jnp.sum(jnp.where(onehot, yt_c, 0.0), axis=1, keepdims=True)
        rpc = jnp.sum(jnp.where(onehot, rpr_s[...], 0.0), axis=1,
                      keepdims=True)
        ypc_s[...] = ypc
        ytc_s[...] = ytc
        gr_s[...] = (ytc - mn) / denom
        dr_s[...] = 1.0 / jnp.log2(rpc + 2.0)

        acc_ref[0] = 0.0
        acc_ref[1] = 0.0
        acc_ref[2] = 0.0
        acc_ref[3] = 0.0
        acc_ref[4] = mn
        acc_ref[5] = denom
        acc_ref[6] = ideal
        acc_ref[7] = listnet - IC_W * corr

    @pl.when(i > nt_rank)
    def _loss():
        r0 = (i - nt_rank - 1) * _TILE_R
        yt_c = yt_ref[...]
        yp_c = yp_ref[...]
        mn = acc_ref[4]
        denom = acc_ref[5]
        g_c = (yt_c - mn) / denom
        d_c = 1.0 / jnp.log2(rpr_s[...] + 2.0)

        yp_r = ypc_s[pl.ds(r0, _TILE_R), :]
        yt_r = ytc_s[pl.ds(r0, _TILE_R), :]
        g_r = gr_s[pl.ds(r0, _TILE_R), :]
        d_r = dr_s[pl.ds(r0, _TILE_R), :]
        gi = r0 + jax.lax.broadcasted_iota(jnp.int32, (_TILE_R, 1), 0)
        top_pair = gi < k_pair
        top_lam = gi < k_lam

        x = yp_c - yp_r
        sp = _softplus(x)
        td = yt_r - yt_c
        pos = td > 0.0
        m_pair = pos & top_pair
        m_lam = pos & top_lam
        acc_ref[0] += jnp.sum(jnp.where(m_pair, sp * td, 0.0))
        acc_ref[1] += jnp.sum(m_pair.astype(jnp.float32))
        delta = jnp.abs((g_r - g_c) * (d_r - d_c))
        acc_ref[2] += jnp.sum(jnp.where(m_lam, sp * delta, 0.0))
        acc_ref[3] += jnp.sum(m_lam.astype(jnp.float32))

        @pl.when(i == nt_rank + nt_loss)
        def _finalize():
            pair_loss = acc_ref[0] / (acc_ref[1] + 1e-8)
            cnt = acc_ref[3]
            lam_loss = jnp.where(
                cnt > 0.0,
                (acc_ref[2] / acc_ref[6]) / jnp.maximum(cnt, 1.0),
                0.0)
            total = acc_ref[7] + pair_loss + lam_loss
            out_ref[...] = jnp.full((1, 1), total, dtype=jnp.float32)


def kernel(y_pred, y_true):
    n = y_pred.shape[1]
    k_pair = max(1, int(n * TOP_FRACTION))
    k_lam = min(LAMBDA_TOPK, n)
    nt_rank = n // _TILE_C
    nc = min(n, -(-k_pair // _TILE_R) * _TILE_R)
    nt_loss = nc // _TILE_R

    yp = y_pred.reshape(1, n)
    yt = y_true.reshape(1, n)
    ypt = y_pred.reshape(n, 1)
    ytt = y_true.reshape(n, 1)

    full_row = pl.BlockSpec((1, n), lambda i: (0, 0))
    full_col = pl.BlockSpec((n, 1), lambda i: (0, 0))

    out = pl.pallas_call(
        functools.partial(_fused_kernel, nt_rank, nt_loss, nc, k_pair, k_lam),
        grid=(nt_rank + 1 + nt_loss,),
        in_specs=[full_row, full_row, full_col, full_col],
        out_specs=pl.BlockSpec((1, 1), lambda i: (0, 0)),
        out_shape=jax.ShapeDtypeStruct((1, 1), jnp.float32),
        scratch_shapes=[
            pltpu.VMEM((1, n), jnp.float32),
            pltpu.VMEM((1, n), jnp.float32),
            pltpu.VMEM((nc, 1), jnp.float32),
            pltpu.VMEM((nc, 1), jnp.float32),
            pltpu.VMEM((nc, 1), jnp.float32),
            pltpu.VMEM((nc, 1), jnp.float32),
            pltpu.SMEM((8,), jnp.float32),
        ],
    )(yp, yt, ypt, ytt)

    return out[0, 0]


# speedup vs baseline: 2.5566x; 1.0320x over previous
import functools

import jax
import jax.numpy as jnp
from jax.experimental import pallas as pl
from jax.experimental.pallas import tpu as pltpu

TEMPERATURE = 10.0
TOP_FRACTION = 0.1
LAMBDA_TOPK = 50
IC_W = 0.5

_TILE_C = 512
_TILE_R = 256
_TILE_L = 64


def _softplus(x):
    return jnp.maximum(x, 0.0) + jnp.log1p(jnp.exp(-jnp.abs(x)))


def _fused_kernel(nt_rank, nt_loss, nc, k_pair, k_lam,
                  yp_ref, yt_ref, ypt_ref, ytt_ref, out_ref,
                  rtr_s, rpr_s, ypc_s, ytc_s, gr_s, dr_s, acc_ref):
    i = pl.program_id(0)
    n = yp_ref.shape[1]

    @pl.when(i < nt_rank)
    def _ranks():
        c0 = i * _TILE_C
        yp_cols = yp_ref[0:1, pl.ds(c0, _TILE_C)]
        yt_cols = yt_ref[0:1, pl.ds(c0, _TILE_C)]
        yp_rows = ypt_ref[...]
        yt_rows = ytt_ref[...]
        row_idx = jax.lax.broadcasted_iota(jnp.int32, (n, 1), 0)
        col_idx = c0 + jax.lax.broadcasted_iota(jnp.int32, (1, _TILE_C), 1)
        earlier = row_idx < col_idx
        cmp_t = (yt_rows > yt_cols) | ((yt_rows == yt_cols) & earlier)
        rtr_s[0:1, pl.ds(c0, _TILE_C)] = jnp.sum(
            cmp_t.astype(jnp.float32), axis=0, keepdims=True)
        cmp_p = (yp_rows > yp_cols) | ((yp_rows == yp_cols) & earlier)
        rpr_s[0:1, pl.ds(c0, _TILE_C)] = jnp.sum(
            cmp_p.astype(jnp.float32), axis=0, keepdims=True)

    @pl.when(i == nt_rank)
    def _compact():
        yt_c = yt_ref[...]
        yp_c = yp_ref[...]
        mn = jnp.min(yt_c)
        mx = jnp.max(yt_c)
        denom = (mx - mn) + 1e-8
        g_c = (yt_c - mn) / denom
        rtr = rtr_s[...]
        ideal = jnp.sum(g_c / jnp.log2(rtr + 2.0)) + 1e-8

        a = yt_c * TEMPERATURE
        e = jnp.exp(a - jnp.max(a))
        p_true = e / jnp.sum(e)
        bm = jnp.max(yp_c)
        logsm = (yp_c - bm) - jnp.log(jnp.sum(jnp.exp(yp_c - bm)))
        listnet = -jnp.sum(p_true * logsm)

        xc = yp_c - jnp.mean(yp_c)
        yc = yt_c - jnp.mean(yt_c)
        xs = jnp.sqrt(jnp.mean(xc * xc) + 1e-12)
        ys = jnp.sqrt(jnp.mean(yc * yc) + 1e-12)
        corr = jnp.clip(jnp.mean(xc * yc) / (xs * ys + 1e-12), -1.0, 1.0)

        r_iota = jax.lax.broadcasted_iota(
            jnp.int32, (nc, 1), 0).astype(jnp.float32)
        onehot = rtr == r_iota
        ypc = jnp.sum(jnp.where(onehot, yp_c, 0.0), axis=1, keepdims=True)
        ytc = jnp.sum(jnp.where(onehot, yt_c, 0.0), axis=1, keepdims=True)
        rpc = jnp.sum(jnp.where(onehot, rpr_s[...], 0.0), axis=1,
                      keepdims=True)
        ypc_s[...] = ypc
        ytc_s[...] = ytc
        gr_s[...] = (ytc - mn) / denom
        dr_s[...] = 1.0 / jnp.log2(rpc + 2.0)

        acc_ref[0] = 0.0
        acc_ref[1] = 0.0
        acc_ref[2] = 0.0
        acc_ref[3] = 0.0
        acc_ref[4] = mn
        acc_ref[5] = denom
        acc_ref[6] = ideal
        acc_ref[7] = listnet - IC_W * corr

    @pl.when(i == nt_rank + 1)
    def _lambda_loss():
        yt_c = yt_ref[...]
        yp_c = yp_ref[...]
        mn = acc_ref[4]
        denom = acc_ref[5]
        g_c = (yt_c - mn) / denom
        d_c = 1.0 / jnp.log2(rpr_s[...] + 2.0)

        yp_r = ypc_s[pl.ds(0, _TILE_L), :]
        yt_r = ytc_s[pl.ds(0, _TILE_L), :]
        g_r = gr_s[pl.ds(0, _TILE_L), :]
        d_r = dr_s[pl.ds(0, _TILE_L), :]
        gi = jax.lax.broadcasted_iota(jnp.int32, (_TILE_L, 1), 0)
        top_lam = gi < k_lam

        x = yp_c - yp_r
        sp = _softplus(x)
        td = yt_r - yt_c
        m_lam = (td > 0.0) & top_lam
        delta = jnp.abs((g_r - g_c) * (d_r - d_c))
        acc_ref[2] = jnp.sum(jnp.where(m_lam, sp * delta, 0.0))
        acc_ref[3] = jnp.sum(m_lam.astype(jnp.float32))

    @pl.when(i > nt_rank + 1)
    def _pair_loss():
        yt_c = yt_ref[...]
        yp_c = yp_ref[...]
        r0 = (i - nt_rank - 2) * _TILE_R
        yp_r = ypc_s[pl.ds(r0, _TILE_R), :]
        yt_r = ytc_s[pl.ds(r0, _TILE_R), :]
        gi = r0 + jax.lax.broadcasted_iota(jnp.int32, (_TILE_R, 1), 0)
        top_pair = gi < k_pair

        x = yp_c - yp_r
        sp = _softplus(x)
        td = yt_r - yt_c
        m_pair = (td > 0.0) & top_pair
        acc_ref[0] += jnp.sum(jnp.where(m_pair, sp * td, 0.0))
        acc_ref[1] += jnp.sum(m_pair.astype(jnp.float32))

        @pl.when(i == nt_rank + 1 + nt_loss)
        def _finalize():
            pair_loss = acc_ref[0] / (acc_ref[1] + 1e-8)
            cnt = acc_ref[3]
            lam_loss = jnp.where(
                cnt > 0.0,
                (acc_ref[2] / acc_ref[6]) / jnp.maximum(cnt, 1.0),
                0.0)
            total = acc_ref[7] + pair_loss + lam_loss
            out_ref[...] = jnp.full((1, 1), total, dtype=jnp.float32)


def kernel(y_pred, y_true):
    n = y_pred.shape[1]
    k_pair = max(1, int(n * TOP_FRACTION))
    k_lam = min(LAMBDA_TOPK, n)
    nt_rank = n // _TILE_C
    nc = min(n, -(-k_pair // _TILE_R) * _TILE_R)
    nt_loss = nc // _TILE_R

    yp = y_pred.reshape(1, n)
    yt = y_true.reshape(1, n)
    ypt = y_pred.reshape(n, 1)
    ytt = y_true.reshape(n, 1)

    full_row = pl.BlockSpec((1, n), lambda i: (0, 0))
    full_col = pl.BlockSpec((n, 1), lambda i: (0, 0))

    assert k_lam <= _TILE_L <= nc
    out = pl.pallas_call(
        functools.partial(_fused_kernel, nt_rank, nt_loss, nc, k_pair, k_lam),
        grid=(nt_rank + 2 + nt_loss,),
        in_specs=[full_row, full_row, full_col, full_col],
        out_specs=pl.BlockSpec((1, 1), lambda i: (0, 0)),
        out_shape=jax.ShapeDtypeStruct((1, 1), jnp.float32),
        scratch_shapes=[
            pltpu.VMEM((1, n), jnp.float32),
            pltpu.VMEM((1, n), jnp.float32),
            pltpu.VMEM((nc, 1), jnp.float32),
            pltpu.VMEM((nc, 1), jnp.float32),
            pltpu.VMEM((nc, 1), jnp.float32),
            pltpu.VMEM((nc, 1), jnp.float32),
            pltpu.SMEM((8,), jnp.float32),
        ],
    )(yp, yt, ypt, ytt)

    return out[0, 0]


# speedup vs baseline: 5.1067x; 1.9975x over previous
import functools

import jax
import jax.numpy as jnp
from jax.experimental import pallas as pl
from jax.experimental.pallas import tpu as pltpu

TEMPERATURE = 10.0
TOP_FRACTION = 0.1
LAMBDA_TOPK = 50
IC_W = 0.5

_SUB = 8
_TILE_R = 256
_TILE_L = 64


def _softplus(x):
    return jnp.maximum(x, 0.0) + jnp.log1p(jnp.exp(-jnp.abs(x)))


def _xor_partner(x, bit_d, d, lanes):
    if d < lanes:
        axis, amt = 1, d
    else:
        axis, amt = 0, d // lanes
    return jnp.where(bit_d, jnp.roll(x, amt, axis=axis),
                     jnp.roll(x, -amt, axis=axis))


def _bitonic(arrs, less_fn, i_lin, n, lanes):
    size = 2
    while size <= n:
        d = size // 2
        while d >= 1:
            bit_d = (i_lin & d) != 0
            partners = [_xor_partner(a, bit_d, d, lanes) for a in arrs]
            own_less = less_fn(arrs, partners)
            bit_s = (i_lin & size) != 0
            want_earlier = bit_d == bit_s
            take_own = own_less == want_earlier
            arrs = [jnp.where(take_own, a, p)
                    for a, p in zip(arrs, partners)]
            d //= 2
        size *= 2
    return arrs


def _desc_val_asc_idx(own, partner):
    v, i = own[0], own[1]
    pv, pi = partner[0], partner[1]
    return (v > pv) | ((v == pv) & (i < pi))


def _asc_key(own, partner):
    return own[0] < partner[0]


def _fused_kernel(n, nt_loss, nc, k_pair, k_lam,
                  yp_ref, yt_ref, yp8_ref, yt8_ref, out_ref,
                  rp8_s, ypc_s, ytc_s, gr_s, dr_s, acc_ref):
    i = pl.program_id(0)
    lanes = n // _SUB

    @pl.when(i == 0)
    def _sorts():
        i_lin = (jax.lax.broadcasted_iota(jnp.int32, (_SUB, lanes), 0) * lanes
                 + jax.lax.broadcasted_iota(jnp.int32, (_SUB, lanes), 1))
        yp8 = yp8_ref[...]
        yt8 = yt8_ref[...]

        _, idx_p = _bitonic([yp8, i_lin], _desc_val_asc_idx, i_lin, n, lanes)
        _, rp8 = _bitonic([idx_p, i_lin], _asc_key, i_lin, n, lanes)
        rp8f = rp8.astype(jnp.float32)
        rp8_s[...] = rp8f
        yt_sort, _, yp_bt, rp_bt = _bitonic(
            [yt8, i_lin, yp8, rp8f], _desc_val_asc_idx, i_lin, n, lanes)

        mn = jnp.min(yt8)
        mx = jnp.max(yt8)
        denom = (mx - mn) + 1e-8
        g_sort = (yt_sort - mn) / denom
        disc = 1.0 / jnp.log2(i_lin.astype(jnp.float32) + 2.0)
        ideal = jnp.sum(g_sort * disc) + 1e-8

        a = yt8 * TEMPERATURE
        e = jnp.exp(a - jnp.max(a))
        p_true = e / jnp.sum(e)
        bm = jnp.max(yp8)
        logsm = (yp8 - bm) - jnp.log(jnp.sum(jnp.exp(yp8 - bm)))
        listnet = -jnp.sum(p_true * logsm)

        xc = yp8 - jnp.mean(yp8)
        yc = yt8 - jnp.mean(yt8)
        xs = jnp.sqrt(jnp.mean(xc * xc) + 1e-12)
        ys = jnp.sqrt(jnp.mean(yc * yc) + 1e-12)
        corr = jnp.clip(jnp.mean(xc * yc) / (xs * ys + 1e-12), -1.0, 1.0)

        rows = nc // lanes
        ytc = yt_sort[0:rows, :].reshape(nc, 1)
        ypc = yp_bt[0:rows, :].reshape(nc, 1)
        rpc = rp_bt[0:rows, :].reshape(nc, 1)
        ytc_s[...] = ytc
        ypc_s[...] = ypc
        gr_s[...] = (ytc[0:_TILE_L] - mn) / denom
        dr_s[...] = 1.0 / jnp.log2(rpc[0:_TILE_L] + 2.0)

        acc_ref[0] = 0.0
        acc_ref[1] = 0.0
        acc_ref[2] = 0.0
        acc_ref[3] = 0.0
        acc_ref[4] = mn
        acc_ref[5] = denom
        acc_ref[6] = ideal
        acc_ref[7] = listnet - IC_W * corr

    @pl.when(i == 1)
    def _lambda_loss():
        yp3 = yp8_ref[...][None]
        yt3 = yt8_ref[...][None]
        mn = acc_ref[4]
        denom = acc_ref[5]
        g3 = (yt3 - mn) / denom
        d3 = 1.0 / jnp.log2(rp8_s[...][None] + 2.0)

        yp_r = ypc_s[0:_TILE_L].reshape(_TILE_L, 1, 1)
        yt_r = ytc_s[0:_TILE_L].reshape(_TILE_L, 1, 1)
        g_r = gr_s[...].reshape(_TILE_L, 1, 1)
        d_r = dr_s[...].reshape(_TILE_L, 1, 1)
        gi = jax.lax.broadcasted_iota(jnp.int32, (_TILE_L, 1, 1), 0)
        top_lam = gi < k_lam

        x = yp3 - yp_r
        sp = _softplus(x)
        td = yt_r - yt3
        m_lam = (td > 0.0) & top_lam
        delta = jnp.abs((g_r - g3) * (d_r - d3))
        acc_ref[2] = jnp.sum(jnp.where(m_lam, sp * delta, 0.0))
        acc_ref[3] = jnp.sum(m_lam.astype(jnp.float32))

    @pl.when(i > 1)
    def _pair_loss():
        yt_c = yt_ref[...]
        yp_c = yp_ref[...]
        r0 = (i - 2) * _TILE_R
        yp_r = ypc_s[pl.ds(r0, _TILE_R), :]
        yt_r = ytc_s[pl.ds(r0, _TILE_R), :]
        gi = r0 + jax.lax.broadcasted_iota(jnp.int32, (_TILE_R, 1), 0)
        top_pair = gi < k_pair

        x = yp_c - yp_r
        sp = _softplus(x)
        td = yt_r - yt_c
        m_pair = (td > 0.0) & top_pair
        acc_ref[0] += jnp.sum(jnp.where(m_pair, sp * td, 0.0))
        acc_ref[1] += jnp.sum(m_pair.astype(jnp.float32))

        @pl.when(i == 1 + nt_loss)
        def _finalize():
            pair_loss = acc_ref[0] / (acc_ref[1] + 1e-8)
            cnt = acc_ref[3]
            lam_loss = jnp.where(
                cnt > 0.0,
                (acc_ref[2] / acc_ref[6]) / jnp.maximum(cnt, 1.0),
                0.0)
            total = acc_ref[7] + pair_loss + lam_loss
            out_ref[...] = jnp.full((1, 1), total, dtype=jnp.float32)


def kernel(y_pred, y_true):
    n = y_pred.shape[1]
    k_pair = max(1, int(n * TOP_FRACTION))
    k_lam = min(LAMBDA_TOPK, n)
    nc = min(n, -(-k_pair // _TILE_R) * _TILE_R)
    nt_loss = nc // _TILE_R
    lanes = n // _SUB
    assert k_lam <= _TILE_L <= nc and nc % lanes == 0 and (n & (n - 1)) == 0

    yp = y_pred.reshape(1, n)
    yt = y_true.reshape(1, n)
    yp8 = y_pred.reshape(_SUB, lanes)
    yt8 = y_true.reshape(_SUB, lanes)

    full_row = pl.BlockSpec((1, n), lambda i: (0, 0))
    full_8 = pl.BlockSpec((_SUB, lanes), lambda i: (0, 0))

    out = pl.pallas_call(
        functools.partial(_fused_kernel, n, nt_loss, nc, k_pair, k_lam),
        grid=(2 + nt_loss,),
        in_specs=[full_row, full_row, full_8, full_8],
        out_specs=pl.BlockSpec((1, 1), lambda i: (0, 0)),
        out_shape=jax.ShapeDtypeStruct((1, 1), jnp.float32),
        scratch_shapes=[
            pltpu.VMEM((_SUB, lanes), jnp.float32),
            pltpu.VMEM((nc, 1), jnp.float32),
            pltpu.VMEM((nc, 1), jnp.float32),
            pltpu.VMEM((_TILE_L, 1), jnp.float32),
            pltpu.VMEM((_TILE_L, 1), jnp.float32),
            pltpu.SMEM((8,), jnp.float32),
        ],
    )(yp, yt, yp8, yt8)

    return out[0, 0]


# speedup vs baseline: 7.1546x; 1.4010x over previous
import functools

import jax
import jax.numpy as jnp
from jax.experimental import pallas as pl
from jax.experimental.pallas import tpu as pltpu

TEMPERATURE = 10.0
TOP_FRACTION = 0.1
LAMBDA_TOPK = 50
IC_W = 0.5

_SUB = 8
_TILE_R = 256
_TILE_L = 64


def _softplus(x):
    return jnp.maximum(x, 0.0) + jnp.log1p(jnp.exp(-jnp.abs(x)))


def _xor_partner(x, bit_d, d, lanes):
    if d < lanes:
        axis, amt = 1, d
    else:
        axis, amt = 0, d // lanes
    return jnp.where(bit_d, jnp.roll(x, amt, axis=axis),
                     jnp.roll(x, -amt, axis=axis))


def _stage(arrs, i_lin, size, d, lanes):
    bit_d = (i_lin & d) != 0
    partners = [_xor_partner(a, bit_d, d, lanes) for a in arrs]
    v, ix = arrs[0], arrs[1]
    pv, pix = partners[0], partners[1]
    own_less = (v > pv) | ((v == pv) & (ix < pix))
    bit_s = (i_lin & size) != 0
    take_own = own_less == (bit_d == bit_s)
    return [jnp.where(take_own, a, p) for a, p in zip(arrs, partners)]


def _fused_kernel(n, nt_loss, nc, k_pair, k_lam,
                  yp_ref, yt_ref, yp8_ref, yt8_ref, out_ref,
                  ypsp_s, ytsp_s, ixsp_s, ypc_s, ytc_s, gr_s, ixr_s,
                  acc_ref):
    i = pl.program_id(0)
    lanes = n // _SUB

    @pl.when(i == 0)
    def _sorts():
        i_lin = (jax.lax.broadcasted_iota(jnp.int32, (_SUB, lanes), 0) * lanes
                 + jax.lax.broadcasted_iota(jnp.int32, (_SUB, lanes), 1))
        yp8 = yp8_ref[...]
        yt8 = yt8_ref[...]

        s_pred = [yp8, i_lin, yt8]
        s_true = [yt8, i_lin, yp8]
        size = 2
        while size <= n:
            d = size // 2
            while d >= 1:
                s_pred = _stage(s_pred, i_lin, size, d, lanes)
                s_true = _stage(s_true, i_lin, size, d, lanes)
                d //= 2
            size *= 2
        yp_sp, ix_sp, yt_sp = s_pred
        yt_bt, ix_bt, yp_bt = s_true
        ypsp_s[...] = yp_sp
        ytsp_s[...] = yt_sp
        ixsp_s[...] = ix_sp

        mn = jnp.min(yt8)
        mx = jnp.max(yt8)
        denom = (mx - mn) + 1e-8
        g_sort = (yt_bt - mn) / denom
        disc = 1.0 / jnp.log2(i_lin.astype(jnp.float32) + 2.0)
        ideal = jnp.sum(g_sort * disc) + 1e-8

        a = yt8 * TEMPERATURE
        e = jnp.exp(a - jnp.max(a))
        p_true = e / jnp.sum(e)
        bm = jnp.max(yp8)
        logsm = (yp8 - bm) - jnp.log(jnp.sum(jnp.exp(yp8 - bm)))
        listnet = -jnp.sum(p_true * logsm)

        xc = yp8 - jnp.mean(yp8)
        yc = yt8 - jnp.mean(yt8)
        xs = jnp.sqrt(jnp.mean(xc * xc) + 1e-12)
        ys = jnp.sqrt(jnp.mean(yc * yc) + 1e-12)
        corr = jnp.clip(jnp.mean(xc * yc) / (xs * ys + 1e-12), -1.0, 1.0)

        rows = nc // lanes
        ytc = yt_bt[0:rows, :].reshape(nc, 1)
        ypc = yp_bt[0:rows, :].reshape(nc, 1)
        ixc = ix_bt[0:rows, :].reshape(nc, 1)
        ytc_s[...] = ytc
        ypc_s[...] = ypc
        gr_s[...] = (ytc[0:_TILE_L] - mn) / denom
        ixr_s[...] = ixc[0:_TILE_L]

        acc_ref[0] = 0.0
        acc_ref[1] = 0.0
        acc_ref[2] = 0.0
        acc_ref[3] = 0.0
        acc_ref[4] = mn
        acc_ref[5] = denom
        acc_ref[6] = ideal
        acc_ref[7] = listnet - IC_W * corr

    @pl.when(i == 1)
    def _lambda_loss():
        i_lin = (jax.lax.broadcasted_iota(jnp.int32, (_SUB, lanes), 0) * lanes
                 + jax.lax.broadcasted_iota(jnp.int32, (_SUB, lanes), 1))
        disc = 1.0 / jnp.log2(i_lin.astype(jnp.float32) + 2.0)
        yp3 = ypsp_s[...][None]
        yt3 = ytsp_s[...][None]
        mn = acc_ref[4]
        denom = acc_ref[5]
        g3 = (yt3 - mn) / denom
        d3 = disc[None]

        yp_r = ypc_s[0:_TILE_L].reshape(_TILE_L, 1, 1)
        yt_r = ytc_s[0:_TILE_L].reshape(_TILE_L, 1, 1)
        g_r = gr_s[...].reshape(_TILE_L, 1, 1)
        onehot = ixr_s[...].reshape(_TILE_L, 1, 1) == ixsp_s[...][None]
        d_r = jnp.sum(jnp.where(onehot, d3, 0.0), axis=(1, 2), keepdims=True)
        gi = jax.lax.broadcasted_iota(jnp.int32, (_TILE_L, 1, 1), 0)
        top_lam = gi < k_lam

        x = yp3 - yp_r
        sp = _softplus(x)
        td = yt_r - yt3
        m_lam = (td > 0.0) & top_lam
        delta = jnp.abs((g_r - g3) * (d_r - d3))
        acc_ref[2] = jnp.sum(jnp.where(m_lam, sp * delta, 0.0))
        acc_ref[3] = jnp.sum(m_lam.astype(jnp.float32))

    @pl.when(i > 1)
    def _pair_loss():
        yt_c = yt_ref[...]
        yp_c = yp_ref[...]
        r0 = (i - 2) * _TILE_R
        yp_r = ypc_s[pl.ds(r0, _TILE_R), :]
        yt_r = ytc_s[pl.ds(r0, _TILE_R), :]
        gi = r0 + jax.lax.broadcasted_iota(jnp.int32, (_TILE_R, 1), 0)
        top_pair = gi < k_pair

        x = yp_c - yp_r
        sp = _softplus(x)
        td = yt_r - yt_c
        m_pair = (td > 0.0) & top_pair
        acc_ref[0] += jnp.sum(jnp.where(m_pair, sp * td, 0.0))
        acc_ref[1] += jnp.sum(m_pair.astype(jnp.float32))

        @pl.when(i == 1 + nt_loss)
        def _finalize():
            pair_loss = acc_ref[0] / (acc_ref[1] + 1e-8)
            cnt = acc_ref[3]
            lam_loss = jnp.where(
                cnt > 0.0,
                (acc_ref[2] / acc_ref[6]) / jnp.maximum(cnt, 1.0),
                0.0)
            total = acc_ref[7] + pair_loss + lam_loss
            out_ref[...] = jnp.full((1, 1), total, dtype=jnp.float32)


def kernel(y_pred, y_true):
    n = y_pred.shape[1]
    k_pair = max(1, int(n * TOP_FRACTION))
    k_lam = min(LAMBDA_TOPK, n)
    nc = min(n, -(-k_pair // _TILE_R) * _TILE_R)
    nt_loss = nc // _TILE_R
    lanes = n // _SUB
    assert k_lam <= _TILE_L <= nc and nc % lanes == 0 and (n & (n - 1)) == 0

    yp = y_pred.reshape(1, n)
    yt = y_true.reshape(1, n)
    yp8 = y_pred.reshape(_SUB, lanes)
    yt8 = y_true.reshape(_SUB, lanes)

    full_row = pl.BlockSpec((1, n), lambda i: (0, 0))
    full_8 = pl.BlockSpec((_SUB, lanes), lambda i: (0, 0))

    out = pl.pallas_call(
        functools.partial(_fused_kernel, n, nt_loss, nc, k_pair, k_lam),
        grid=(2 + nt_loss,),
        in_specs=[full_row, full_row, full_8, full_8],
        out_specs=pl.BlockSpec((1, 1), lambda i: (0, 0)),
        out_shape=jax.ShapeDtypeStruct((1, 1), jnp.float32),
        scratch_shapes=[
            pltpu.VMEM((_SUB, lanes), jnp.float32),
            pltpu.VMEM((_SUB, lanes), jnp.float32),
            pltpu.VMEM((_SUB, lanes), jnp.int32),
            pltpu.VMEM((nc, 1), jnp.float32),
            pltpu.VMEM((nc, 1), jnp.float32),
            pltpu.VMEM((_TILE_L, 1), jnp.float32),
            pltpu.VMEM((_TILE_L, 1), jnp.int32),
            pltpu.SMEM((8,), jnp.float32),
        ],
    )(yp, yt, yp8, yt8)

    return out[0, 0]


# speedup vs baseline: 8.6407x; 1.2077x over previous
import functools

import jax
import jax.numpy as jnp
from jax.experimental import pallas as pl
from jax.experimental.pallas import tpu as pltpu

TEMPERATURE = 10.0
TOP_FRACTION = 0.1
LAMBDA_TOPK = 50
IC_W = 0.5

_SUB = 8
_TILE_R = 256
_TILE_L = 64


def _softplus(x):
    return jnp.maximum(x, 0.0) + jnp.log1p(jnp.exp(-jnp.abs(x)))


def _xor_partner(x, bit_d, d, lanes):
    if d < lanes:
        axis, amt = 1, d
    else:
        axis, amt = 0, d // lanes
    return jnp.where(bit_d, jnp.roll(x, amt, axis=axis),
                     jnp.roll(x, -amt, axis=axis))


def _stage(arrs, i_lin, size, d, lanes):
    bit_d = (i_lin & d) != 0
    partners = [_xor_partner(a, bit_d, d, lanes) for a in arrs]
    v, ix = arrs[0], arrs[1]
    pv, pix = partners[0], partners[1]
    own_less = (v > pv) | ((v == pv) & (ix < pix))
    bit_s = (i_lin & size) != 0
    take_own = own_less == (bit_d == bit_s)
    return [jnp.where(take_own, a, p) for a, p in zip(arrs, partners)]


def _fused_kernel(n, nc, k_pair, k_lam, yp_ref, yt_ref, out_ref):
    lanes = n // _SUB
    i_lin = (jax.lax.broadcasted_iota(jnp.int32, (_SUB, lanes), 0) * lanes
             + jax.lax.broadcasted_iota(jnp.int32, (_SUB, lanes), 1))
    yp8 = yp_ref[...]
    yt8 = yt_ref[...]

    s_pred = [yp8, i_lin, yt8]
    s_true = [yt8, i_lin, yp8]
    size = 2
    while size <= n:
        d = size // 2
        while d >= 1:
            s_pred = _stage(s_pred, i_lin, size, d, lanes)
            s_true = _stage(s_true, i_lin, size, d, lanes)
            d //= 2
        size *= 2
    yp_sp, ix_sp, yt_sp = s_pred
    yt_bt, ix_bt, yp_bt = s_true

    mn = jnp.min(yt8)
    mx = jnp.max(yt8)
    denom = (mx - mn) + 1e-8
    disc = 1.0 / jnp.log2(i_lin.astype(jnp.float32) + 2.0)
    ideal = jnp.sum(((yt_bt - mn) / denom) * disc) + 1e-8

    a = yt8 * TEMPERATURE
    e = jnp.exp(a - jnp.max(a))
    p_true = e / jnp.sum(e)
    bm = jnp.max(yp8)
    logsm = (yp8 - bm) - jnp.log(jnp.sum(jnp.exp(yp8 - bm)))
    listnet = -jnp.sum(p_true * logsm)

    xc = yp8 - jnp.mean(yp8)
    yc = yt8 - jnp.mean(yt8)
    xs = jnp.sqrt(jnp.mean(xc * xc) + 1e-12)
    ys = jnp.sqrt(jnp.mean(yc * yc) + 1e-12)
    corr = jnp.clip(jnp.mean(xc * yc) / (xs * ys + 1e-12), -1.0, 1.0)

    rows = nc // lanes
    ytc = yt_bt[0:rows, :].reshape(nc, 1)
    ypc = yp_bt[0:rows, :].reshape(nc, 1)
    ixc = ix_bt[0:rows, :].reshape(nc, 1)

    yp3 = yp_sp[None]
    yt3 = yt_sp[None]
    g3 = (yt3 - mn) / denom
    d3 = disc[None]
    yp_rl = ypc[0:_TILE_L].reshape(_TILE_L, 1, 1)
    yt_rl = ytc[0:_TILE_L].reshape(_TILE_L, 1, 1)
    g_rl = (yt_rl - mn) / denom
    onehot = ixc[0:_TILE_L].reshape(_TILE_L, 1, 1) == ix_sp[None]
    d_rl = jnp.sum(jnp.where(onehot, d3, 0.0), axis=(1, 2), keepdims=True)
    top_lam = jax.lax.broadcasted_iota(jnp.int32, (_TILE_L, 1, 1), 0) < k_lam

    xl = yp3 - yp_rl
    spl = _softplus(xl)
    tdl = yt_rl - yt3
    m_lam = (tdl > 0.0) & top_lam
    delta = jnp.abs((g_rl - g3) * (d_rl - d3))
    lam_num = jnp.sum(jnp.where(m_lam, spl * delta, 0.0))
    lam_cnt = jnp.sum(m_lam.astype(jnp.float32))

    pair_num = 0.0
    pair_cnt = 0.0
    for r0 in range(0, nc, _TILE_R):
        yp_r = ypc[r0:r0 + _TILE_R].reshape(_TILE_R, 1, 1)
        yt_r = ytc[r0:r0 + _TILE_R].reshape(_TILE_R, 1, 1)
        gi = r0 + jax.lax.broadcasted_iota(jnp.int32, (_TILE_R, 1, 1), 0)
        top_pair = gi < k_pair
        x = yp8[None] - yp_r
        sp = _softplus(x)
        td = yt_r - yt8[None]
        m_pair = (td > 0.0) & top_pair
        pair_num += jnp.sum(jnp.where(m_pair, sp * td, 0.0))
        pair_cnt += jnp.sum(m_pair.astype(jnp.float32))

    pair_loss = pair_num / (pair_cnt + 1e-8)
    lam_loss = jnp.where(
        lam_cnt > 0.0, (lam_num / ideal) / jnp.maximum(lam_cnt, 1.0), 0.0)
    total = listnet - IC_W * corr + pair_loss + lam_loss
    out_ref[...] = jnp.full((1, 1), total, dtype=jnp.float32)


def kernel(y_pred, y_true):
    n = y_pred.shape[1]
    k_pair = max(1, int(n * TOP_FRACTION))
    k_lam = min(LAMBDA_TOPK, n)
    nc = min(n, -(-k_pair // _TILE_R) * _TILE_R)
    lanes = n // _SUB
    assert k_lam <= _TILE_L <= nc and nc % lanes == 0 and (n & (n - 1)) == 0

    yp8 = y_pred.reshape(_SUB, lanes)
    yt8 = y_true.reshape(_SUB, lanes)

    out = pl.pallas_call(
        functools.partial(_fused_kernel, n, nc, k_pair, k_lam),
        out_shape=jax.ShapeDtypeStruct((1, 1), jnp.float32),
    )(yp8, yt8)

    return out[0, 0]


# speedup vs baseline: 9.0683x; 1.0495x over previous
import functools

import jax
import jax.numpy as jnp
from jax.experimental import pallas as pl
from jax.experimental.pallas import tpu as pltpu

TEMPERATURE = 10.0
TOP_FRACTION = 0.1
LAMBDA_TOPK = 50
IC_W = 0.5

_SUB = 8
_TILE_L = 56


def _softplus(x):
    return jnp.maximum(x, 0.0) + jnp.log1p(jnp.exp(-jnp.abs(x)))


def _xor_partner(x, bit_d, d, lanes):
    if d < lanes:
        axis, amt = 1, d
    else:
        axis, amt = 0, d // lanes
    return jnp.where(bit_d, jnp.roll(x, amt, axis=axis),
                     jnp.roll(x, -amt, axis=axis))


def _stage(arrs, i_lin, off, size, d, lanes):
    bit_d = ((i_lin + off) & d) != 0
    partners = [_xor_partner(a, bit_d, d, lanes) for a in arrs]
    v, ix = arrs[0], arrs[1]
    pv, pix = partners[0], partners[1]
    own_less = (v > pv) | ((v == pv) & (ix < pix))
    bit_s = ((i_lin + off) & size) != 0
    take_own = own_less == (bit_d == bit_s)
    return [jnp.where(take_own, a, p) for a, p in zip(arrs, partners)]


def _cross_stage(lo, hi):
    v, ix = lo[0], lo[1]
    pv, pix = hi[0], hi[1]
    lo_less = (v > pv) | ((v == pv) & (ix < pix))
    new_lo = [jnp.where(lo_less, a, b) for a, b in zip(lo, hi)]
    new_hi = [jnp.where(lo_less, b, a) for a, b in zip(lo, hi)]
    return new_lo, new_hi


def _fused_kernel(n, nc, k_pair, k_lam, yp_ref, yt_ref, out_ref):
    lanes = n // _SUB
    lanes_h = lanes // 2
    half = n // 2
    i_lin = (jax.lax.broadcasted_iota(
        jnp.int32, (_SUB, lanes_h), 0) * lanes_h
        + jax.lax.broadcasted_iota(jnp.int32, (_SUB, lanes_h), 1))
    yp8 = yp_ref[...]
    yt8 = yt_ref[...]
    yp_h = [yp8[:, 0:lanes_h], yp8[:, lanes_h:lanes]]
    yt_h = [yt8[:, 0:lanes_h], yt8[:, lanes_h:lanes]]

    s_pred = [[yp_h[h], i_lin + h * half, yt_h[h]] for h in (0, 1)]
    s_true = [[yt_h[h], i_lin + h * half, yp_h[h]] for h in (0, 1)]
    size = 2
    while size <= n:
        d = size // 2
        while d >= 1:
            if d == half:
                s_pred = list(_cross_stage(*s_pred))
                s_true = list(_cross_stage(*s_true))
            else:
                for h, off in ((0, 0), (1, half)):
                    s_pred[h] = _stage(s_pred[h], i_lin, off, size, d,
                                       lanes_h)
                    s_true[h] = _stage(s_true[h], i_lin, off, size, d,
                                       lanes_h)
            d //= 2
        size *= 2
    (yp_sp0, ix_sp0, yt_sp0), (yp_sp1, ix_sp1, yt_sp1) = s_pred
    (yt_bt0, ix_bt0, yp_bt0), (yt_bt1, ix_bt1, yp_bt1) = s_true

    mn = jnp.min(yt8)
    mx = jnp.max(yt8)
    denom = (mx - mn) + 1e-8
    disc0 = 1.0 / jnp.log2(i_lin.astype(jnp.float32) + 2.0)
    disc1 = 1.0 / jnp.log2(i_lin.astype(jnp.float32) + (2.0 + half))
    ideal = (jnp.sum(((yt_bt0 - mn) / denom) * disc0)
             + jnp.sum(((yt_bt1 - mn) / denom) * disc1) + 1e-8)

    a = yt8 * TEMPERATURE
    e = jnp.exp(a - jnp.max(a))
    p_true = e / jnp.sum(e)
    bm = jnp.max(yp8)
    logsm = (yp8 - bm) - jnp.log(jnp.sum(jnp.exp(yp8 - bm)))
    listnet = -jnp.sum(p_true * logsm)

    xc = yp8 - jnp.mean(yp8)
    yc = yt8 - jnp.mean(yt8)
    xs = jnp.sqrt(jnp.mean(xc * xc) + 1e-12)
    ys = jnp.sqrt(jnp.mean(yc * yc) + 1e-12)
    corr = jnp.clip(jnp.mean(xc * yc) / (xs * ys + 1e-12), -1.0, 1.0)

    rows = nc // lanes_h

    def _col(arr):
        return jnp.concatenate(
            [arr[r:r + 1, :].reshape(lanes_h, 1) for r in range(rows)],
            axis=0)

    ytc = _col(yt_bt0)
    ypc = _col(yp_bt0)
    ixc = _col(ix_bt0)

    yp_rl = ypc[0:_TILE_L].reshape(_TILE_L, 1, 1)
    yt_rl = ytc[0:_TILE_L].reshape(_TILE_L, 1, 1)
    g_rl = (yt_rl - mn) / denom
    ix_rl = ixc[0:_TILE_L].reshape(_TILE_L, 1, 1)
    top_lam = jax.lax.broadcasted_iota(jnp.int32, (_TILE_L, 1, 1), 0) < k_lam

    d_rl = 0.0
    for ix_sp, d3 in ((ix_sp0, disc0), (ix_sp1, disc1)):
        onehot = ix_rl == ix_sp[None]
        d_rl += jnp.sum(jnp.where(onehot, d3[None], 0.0), axis=(1, 2),
                        keepdims=True)

    lam_num = 0.0
    lam_cnt = 0.0
    for yp_sp, yt_sp, d3 in ((yp_sp0, yt_sp0, disc0),
                             (yp_sp1, yt_sp1, disc1)):
        yp3 = yp_sp[None]
        yt3 = yt_sp[None]
        g3 = (yt3 - mn) / denom
        xl = yp3 - yp_rl
        spl = _softplus(xl)
        tdl = yt_rl - yt3
        m_lam = (tdl > 0.0) & top_lam
        delta = jnp.abs((g_rl - g3) * (d_rl - d3[None]))
        lam_num += jnp.sum(jnp.where(m_lam, spl * delta, 0.0))
        lam_cnt += jnp.sum(m_lam.astype(jnp.float32))

    pair_num = 0.0
    pair_cnt = 0.0
    tile_r = nc // 2
    for r0 in range(0, nc, tile_r):
        yp_r = ypc[r0:r0 + tile_r].reshape(tile_r, 1, 1)
        yt_r = ytc[r0:r0 + tile_r].reshape(tile_r, 1, 1)
        x = yp8[None] - yp_r
        sp = _softplus(x)
        td = yt_r - yt8[None]
        m_pair = td > 0.0
        if r0 + tile_r > k_pair:
            gi = r0 + jax.lax.broadcasted_iota(jnp.int32, (tile_r, 1, 1), 0)
            m_pair = m_pair & (gi < k_pair)
        pair_num += jnp.sum(jnp.where(m_pair, sp * td, 0.0))
        pair_cnt += jnp.sum(m_pair.astype(jnp.float32))

    pair_loss = pair_num / (pair_cnt + 1e-8)
    lam_loss = jnp.where(
        lam_cnt > 0.0, (lam_num / ideal) / jnp.maximum(lam_cnt, 1.0), 0.0)
    total = listnet - IC_W * corr + pair_loss + lam_loss
    out_ref[...] = jnp.full((1, 1), total, dtype=jnp.float32)


def kernel(y_pred, y_true):
    n = y_pred.shape[1]
    k_pair = max(1, int(n * TOP_FRACTION))
    k_lam = min(LAMBDA_TOPK, n)
    nc = min(n, -(-k_pair // 256) * 256)
    lanes = n // _SUB
    assert k_lam <= _TILE_L <= nc and (n & (n - 1)) == 0 and nc % 16 == 0
    assert nc <= (n // 2) // 2

    yp8 = y_pred.reshape(_SUB, lanes)
    yt8 = y_true.reshape(_SUB, lanes)

    out = pl.pallas_call(
        functools.partial(_fused_kernel, n, nc, k_pair, k_lam),
        out_shape=jax.ShapeDtypeStruct((1, 1), jnp.float32),
    )(yp8, yt8)

    return out[0, 0]
